# Initial kernel scaffold; baseline (speedup 1.0000x reference)
import jax
import jax.numpy as jnp
from jax.experimental import pallas as pl


def kernel(orin_out, labels):
    raise NotImplementedError("write your pallas kernel here")


# speedup vs baseline: 1.0649x; 1.0649x over previous
import jax
import jax.numpy as jnp
from jax.experimental import pallas as pl

_MARGIN_S = 64.0
_MARGIN_M = 0.35
_N = 100000
_B = 1024
_R = 16


def _margin_block(lbl_ref, x_ref, o_ref):
    lbl = lbl_ref[:, 0]
    cols = jax.lax.broadcasted_iota(jnp.int32, (_R, _N), 1)
    mask = cols == lbl[:, None]
    x = x_ref[...]
    o_ref[...] = (x - jnp.where(mask, _MARGIN_M, 0.0)) * _MARGIN_S


def kernel(orin_out, labels):
    lbl2d = labels.astype(jnp.int32).reshape(_B, 1)
    return pl.pallas_call(
        _margin_block,
        grid=(_B // _R,),
        in_specs=[
            pl.BlockSpec((_R, 1), lambda i: (i, 0)),
            pl.BlockSpec((_R, _N), lambda i: (i, 0)),
        ],
        out_specs=pl.BlockSpec((_R, _N), lambda i: (i, 0)),
        out_shape=jax.ShapeDtypeStruct((_B, _N), jnp.float32),
    )(lbl2d, orin_out)
